# Initial kernel scaffold; baseline (speedup 1.0000x reference)
import jax
import jax.numpy as jnp
from jax.experimental import pallas as pl


def kernel(features, coords):
    raise NotImplementedError("write your pallas kernel here")


# speedup vs baseline: 1.3475x; 1.3475x over previous
import functools

import jax
import jax.numpy as jnp
from jax import lax
from jax.experimental import pallas as pl
from jax.experimental.pallas import tpu as pltpu
from jax.experimental.pallas import tpu_sc as plsc

_R = 32
_B, _C, _N = 16, 64, 32768
_R3 = _R * _R * _R
_L = 16
_HALF = _N // 2


def _norm_body(coords_ref, norm_ref, idx_ref):
    c = coords_ref[0]
    mean = jnp.mean(c, axis=1, keepdims=True)
    cc = c - mean
    nrm2 = jnp.sum(cc * cc, axis=0, keepdims=True)
    denom = jnp.sqrt(jnp.max(nrm2)) * 2.0
    nc = cc / denom + 0.5
    nc = jnp.clip(nc * float(_R), 0.0, float(_R - 1))
    norm_ref[0] = nc
    v = jnp.round(nc).astype(jnp.int32)
    idx_ref[0, 0] = v[0] * (_R * _R) + v[1] * _R + v[2]


def _normalize(coords):
    return pl.pallas_call(
        _norm_body,
        grid=(_B,),
        in_specs=[pl.BlockSpec((1, 3, _N), lambda b: (b, 0, 0))],
        out_specs=[
            pl.BlockSpec((1, 3, _N), lambda b: (b, 0, 0)),
            pl.BlockSpec((1, 1, _N), lambda b: (b, 0, 0)),
        ],
        out_shape=[
            jax.ShapeDtypeStruct((_B, 3, _N), jnp.float32),
            jax.ShapeDtypeStruct((_B, 1, _N), jnp.int32),
        ],
    )(coords)


def _scatter_body(feat_hbm, idx_hbm, out_hbm, idx_v, inv_v, acc_v, feat_v):
    b = lax.axis_index("s")
    h = lax.axis_index("c")
    nvec = _N // _L

    pltpu.sync_copy(idx_hbm.at[pl.ds(pl.multiple_of(b * _N, _N), _N)], idx_v)

    ones = jnp.full((_L,), 1.0, jnp.float32)
    zeros = jnp.zeros((_L,), jnp.float32)

    def zero_body(i, _):
        acc_v[pl.ds(i * _L, _L)] = zeros
        return 0

    def cnt_body(i, _):
        vi = idx_v[pl.ds(i * _L, _L)]
        plsc.addupdate_scatter(acc_v, [vi], ones)
        return 0

    def inv_body(i, _):
        s = pl.ds(i * _L, _L)
        inv_v[s] = 1.0 / jnp.maximum(acc_v[s], 1.0)
        return 0

    lax.fori_loop(0, nvec, zero_body, 0)
    lax.fori_loop(0, nvec, cnt_body, 0)
    lax.fori_loop(0, nvec, inv_body, 0)

    def ch_body(ch, _):
        c_abs = h * (_C // 2) + ch
        row = pl.multiple_of((b * _C + c_abs) * _N, _N)
        lax.fori_loop(0, nvec, zero_body, 0)

        def half_body(half, _):
            off = pl.multiple_of(row + half * _HALF, _HALF)
            pltpu.sync_copy(feat_hbm.at[pl.ds(off, _HALF)], feat_v)

            def sc_body(i, _):
                v = feat_v[pl.ds(i * _L, _L)]
                vi = idx_v[pl.ds(half * _HALF + i * _L, _L)]
                plsc.addupdate_scatter(acc_v, [vi], v)
                return 0

            lax.fori_loop(0, _HALF // _L, sc_body, 0)
            return 0

        lax.fori_loop(0, 2, half_body, 0)

        def mul_body(i, _):
            s = pl.ds(i * _L, _L)
            acc_v[s] = acc_v[s] * inv_v[s]
            return 0

        lax.fori_loop(0, nvec, mul_body, 0)
        pltpu.sync_copy(acc_v, out_hbm.at[pl.ds(row, _N)])
        return 0

    lax.fori_loop(0, _C // 2, ch_body, 0)


@functools.cache
def _scatter_call():
    return pl.kernel(
        _scatter_body,
        out_type=jax.ShapeDtypeStruct((_B * _C * _R3,), jnp.float32),
        mesh=plsc.VectorSubcoreMesh(
            core_axis_name="c", subcore_axis_name="s", num_cores=2, num_subcores=16
        ),
        compiler_params=pltpu.CompilerParams(needs_layout_passes=False),
        scratch_types=[
            pltpu.VMEM((_N,), jnp.int32),
            pltpu.VMEM((_R3,), jnp.float32),
            pltpu.VMEM((_R3,), jnp.float32),
            pltpu.VMEM((_HALF,), jnp.float32),
        ],
    )


def kernel(features, coords):
    norm, idx3 = _normalize(coords)
    vox_flat = _scatter_call()(features.reshape(-1), idx3.reshape(-1))
    voxels = vox_flat.reshape(_B, _C, _R, _R, _R)
    return voxels, norm


# speedup vs baseline: 2.3322x; 1.7308x over previous
import functools

import jax
import jax.numpy as jnp
from jax import lax
from jax.experimental import pallas as pl
from jax.experimental.pallas import tpu as pltpu
from jax.experimental.pallas import tpu_sc as plsc

_R = 32
_B, _C, _N = 16, 64, 32768
_R3 = _R * _R * _R
_L = 16
_CH = 8192
_NCHUNK = _N // _CH
_CG = 8


def _norm_body(coords_ref, norm_ref, idx_ref):
    c = coords_ref[0]
    mean = jnp.mean(c, axis=1, keepdims=True)
    cc = c - mean
    nrm2 = jnp.sum(cc * cc, axis=0, keepdims=True)
    denom = jnp.sqrt(jnp.max(nrm2)) * 2.0
    nc = cc / denom + 0.5
    nc = jnp.clip(nc * float(_R), 0.0, float(_R - 1))
    norm_ref[0] = nc
    v = jnp.round(nc).astype(jnp.int32)
    idx_ref[...] = v[0] * (_R * _R) + v[1] * _R + v[2]


def _normalize(coords):
    return pl.pallas_call(
        _norm_body,
        grid=(_B,),
        in_specs=[pl.BlockSpec((1, 3, _N), lambda b: (b, 0, 0))],
        out_specs=[
            pl.BlockSpec((1, 3, _N), lambda b: (b, 0, 0)),
            pl.BlockSpec((_N,), lambda b: (b,)),
        ],
        out_shape=[
            jax.ShapeDtypeStruct((_B, 3, _N), jnp.float32),
            jax.ShapeDtypeStruct((_B * _N,), jnp.int32),
        ],
    )(coords)


def _detile_body(feat_ref, flat_ref):
    flat_ref[...] = feat_ref[0].reshape(-1)


def _detile(features):
    return pl.pallas_call(
        _detile_body,
        grid=(_B, _C // _CG),
        in_specs=[pl.BlockSpec((1, _CG, _N), lambda b, g: (b, g, 0))],
        out_specs=pl.BlockSpec((_CG * _N,), lambda b, g: (b * (_C // _CG) + g,)),
        out_shape=jax.ShapeDtypeStruct((_B * _C * _N,), jnp.float32),
    )(features)


def _scatter_body(feat_hbm, idx_hbm, out_hbm,
                  idx_v, invg_v, acc_v, feat0_v, feat1_v,
                  sem_a, sem_b, sem_out):
    b = lax.axis_index("s")
    h = lax.axis_index("c")

    pltpu.sync_copy(idx_hbm.at[pl.ds(pl.multiple_of(b * _N, _N), _N)], idx_v)

    ones = jnp.full((_L,), 1.0, jnp.float32)
    zeros = jnp.zeros((_L,), jnp.float32)
    feats = (feat0_v, feat1_v)
    feat_sems = (sem_a, sem_b)

    def zero_acc():
        @plsc.parallel_loop(0, _R3, _L, unroll=8)
        def _(i):
            acc_v[pl.ds(i, _L)] = zeros

    zero_acc()

    @plsc.parallel_loop(0, _N, _L, unroll=8)
    def _(i):
        vi = idx_v[pl.ds(i, _L)]
        plsc.addupdate_scatter(acc_v, [vi], ones)

    @plsc.parallel_loop(0, _N, _L, unroll=8)
    def _(i):
        vi = idx_v[pl.ds(i, _L)]
        cnt = plsc.load_gather(acc_v, [vi])
        invg_v[pl.ds(i, _L)] = 1.0 / jnp.maximum(cnt, 1.0)

    def ch_body(ch, _):
        c_abs = h * (_C // 2) + ch
        row = pl.multiple_of((b * _C + c_abs) * _N, _N)

        @pl.when(ch > 0)
        def _():
            pltpu.make_async_copy(
                acc_v, out_hbm.at[pl.ds(row, _N)], sem_out
            ).wait()

        zero_acc()

        pltpu.async_copy(
            feat_hbm.at[pl.ds(row, _CH)], feats[0], feat_sems[0]
        )
        for k in range(_NCHUNK):
            fp = k % 2
            if k + 1 < _NCHUNK:
                off = pl.multiple_of(row + (k + 1) * _CH, _CH)
                pltpu.async_copy(
                    feat_hbm.at[pl.ds(off, _CH)],
                    feats[(k + 1) % 2],
                    feat_sems[(k + 1) % 2],
                )
            pltpu.make_async_copy(
                feat_hbm.at[pl.ds(row, _CH)], feats[fp], feat_sems[fp]
            ).wait()
            base = k * _CH

            @plsc.parallel_loop(0, _CH, _L, unroll=8)
            def _(i):
                v = feats[fp][pl.ds(i, _L)]
                vi = idx_v[pl.ds(base + i, _L)]
                g = invg_v[pl.ds(base + i, _L)]
                plsc.addupdate_scatter(acc_v, [vi], v * g)

        pltpu.async_copy(acc_v, out_hbm.at[pl.ds(row, _N)], sem_out)
        return 0

    lax.fori_loop(0, _C // 2, ch_body, 0)

    last_row = pl.multiple_of((b * _C + h * (_C // 2) + _C // 2 - 1) * _N, _N)
    pltpu.make_async_copy(acc_v, out_hbm.at[pl.ds(last_row, _N)], sem_out).wait()


@functools.cache
def _scatter_call():
    return pl.kernel(
        _scatter_body,
        out_type=jax.ShapeDtypeStruct((_B * _C * _R3,), jnp.float32),
        mesh=plsc.VectorSubcoreMesh(
            core_axis_name="c", subcore_axis_name="s", num_cores=2, num_subcores=16
        ),
        compiler_params=pltpu.CompilerParams(needs_layout_passes=False),
        scratch_types=[
            pltpu.VMEM((_N,), jnp.int32),
            pltpu.VMEM((_N,), jnp.float32),
            pltpu.VMEM((_R3,), jnp.float32),
            pltpu.VMEM((_CH,), jnp.float32),
            pltpu.VMEM((_CH,), jnp.float32),
            pltpu.SemaphoreType.DMA,
            pltpu.SemaphoreType.DMA,
            pltpu.SemaphoreType.DMA,
        ],
    )


def kernel(features, coords):
    norm, idx_flat = _normalize(coords)
    feat_flat = _detile(features)
    vox_flat = _scatter_call()(feat_flat, idx_flat)
    voxels = vox_flat.reshape(_B, _C, _R, _R, _R)
    return voxels, norm
